# baseline (speedup 1.0000x reference)
import functools

import jax
import jax.numpy as jnp
import numpy as np
from jax import lax
from jax.experimental import pallas as pl
from jax.experimental.pallas import tpu as pltpu
from jax.experimental.pallas import tpu_sc as plsc

_FIELD_DIMS = [100000] * 26
_EMBED_DIM = 32
_TOTAL = int(sum(_FIELD_DIMS))
_OFFSETS = np.array((0, *np.cumsum(_FIELD_DIMS)[:-1]), dtype=np.int32)
_BATCH = 4096
_F = len(_FIELD_DIMS)

_NW = 32
_BPW = _BATCH // _NW
_GROUPS = 8
_GB = 16
_SUB = 4
_ROWS_SUB = _GB * _F // _SUB
_ROWS_GRP = _GB * _F
_STEPS = _GROUPS * _SUB


def _body(idx_hbm, bias_hbm, wemb_hbm, wfc_hbm, out_hbm,
          idxf, fcv, ebuf, outv, biasv, sem_e, sem_fc):
    nc = 2
    wid = lax.axis_index("s") * nc + lax.axis_index("c")

    pltpu.sync_copy(idx_hbm.at[wid], idxf)
    pltpu.sync_copy(bias_hbm, biasv)

    for t in range(_STEPS):
        pltpu.async_copy(
            wfc_hbm.at[idxf.at[pl.ds(t * _ROWS_SUB, _ROWS_SUB)]],
            fcv.at[pl.ds(t * _ROWS_SUB, _ROWS_SUB)], sem_fc)

    def _enqueue_group(g, buf):
        def enq(vv, carry):
            off = _ROWS_GRP * g + 16 * vv
            vec = idxf[pl.ds(pl.multiple_of(off, 16), 16)]
            for k in range(16):
                r = vec[k]
                side = lax.shift_right_logical(r, 10) & 1
                p = lax.shift_left(lax.shift_right_logical(r, 11), 10) | (r & 1023)
                cs = pl.multiple_of(side * 64, 64)
                pltpu.async_copy(wemb_hbm.at[p, pl.ds(cs, 64)],
                                 ebuf.at[buf, 16 * vv + k], sem_e)
            return carry
        lax.fori_loop(0, _ROWS_GRP // 16, enq, 0)

    def _drain_group(buf):
        pltpu.make_async_copy(wemb_hbm.at[pl.ds(0, _ROWS_GRP), pl.ds(0, 64)],
                              ebuf.at[buf], sem_e).wait()

    _enqueue_group(0, 0)
    _enqueue_group(1, 1)

    lane = lax.broadcasted_iota(jnp.int32, (16,), 0)

    def _compute_group(g, buf):
        def batch_body(bb, ycarry):
            base = _F * bb
            z = jnp.zeros((16,), jnp.float32)
            s1 = [z, z, z, z]
            s2 = [z, z, z, z]
            s3 = [z, z]
            for f in range(_F):
                for c in range(4):
                    v = ebuf[buf, base + f, pl.ds(16 * c, 16)]
                    s1[c] = s1[c] + v
                    t = v * v
                    s2[c] = s2[c] + t
                    if c >= 2:
                        s3[c - 2] = s3[c - 2] + t * v
            g01 = (s1[0] * s1[0] - s2[0]) + (s1[1] * s1[1] - s2[1])
            h = None
            for c in (2, 3):
                p1, p2, p3 = s1[c], s2[c], s3[c - 2]
                hc = p1 * (p1 * p1 - 3.0 * p2) + 2.0 * p3
                h = hc if h is None else h + hc
            tvec = 0.5 * g01 + (1.0 / 6.0) * h
            y = jnp.sum(tvec)
            ybc = jnp.broadcast_to(y, (16,))
            return jnp.where(lane == bb, ybc, ycarry)
        yg = lax.fori_loop(0, _GB, batch_body, jnp.zeros((16,), jnp.float32))
        outv[pl.ds(pl.multiple_of(_GB * g, 16), _GB)] = yg

    def outer(i, carry):
        g0 = 2 * i
        for buf in range(2):
            g = g0 + buf
            _drain_group(buf)
            _compute_group(g, buf)

            @pl.when(g + 2 < _GROUPS)
            def _():
                _enqueue_group(g + 2, buf)
        return carry

    lax.fori_loop(0, _GROUPS // 2, outer, 0)

    for t in range(_STEPS):
        pltpu.make_async_copy(
            wfc_hbm.at[idxf.at[pl.ds(0, _ROWS_SUB)]],
            fcv.at[pl.ds(t * _ROWS_SUB, _ROWS_SUB)], sem_fc).wait()

    i26 = lax.broadcasted_iota(jnp.int32, (16,), 0) * _F
    for g in range(_GROUPS):
        acc = biasv[...]
        for f in range(_F):
            acc = acc + plsc.load_gather(fcv, [i26 + (_ROWS_GRP * g + f)])
        v = outv[pl.ds(_GB * g, _GB)] + acc
        outv[pl.ds(_GB * g, _GB)] = jnp.maximum(v, 0.0)

    pltpu.sync_copy(outv, out_hbm.at[pl.ds(wid * _BPW, _BPW)])


@functools.partial(jax.jit, static_argnums=())
def kernel(x, W_emb, W_fc, bias):
    offsets = jnp.asarray(_OFFSETS, dtype=jnp.int32)
    idx = (x.astype(jnp.int32) + offsets[None, :]).reshape(
        _NW, _STEPS * _ROWS_SUB)
    bias16 = jnp.broadcast_to(bias.astype(jnp.float32), (16,))
    wfc_flat = W_fc.reshape(_TOTAL)

    wembT = W_emb.T
    _BC = 1024
    _GRID = (_TOTAL + 2 * _BC - 1) // (2 * _BC)

    def _tr_body(a_ref, b_ref, o_ref):
        eye = jnp.eye(64, dtype=jnp.float32)
        dn = (((0,), (0,)), ((), ()))
        ya = lax.dot_general(a_ref[...], eye, dn,
                             precision=lax.Precision.HIGHEST,
                             preferred_element_type=jnp.float32)
        yb = lax.dot_general(b_ref[...], eye, dn,
                             precision=lax.Precision.HIGHEST,
                             preferred_element_type=jnp.float32)
        o_ref[...] = jnp.concatenate([ya, yb], axis=1)

    w2 = pl.pallas_call(
        _tr_body,
        grid=(_GRID,),
        in_specs=[pl.BlockSpec((64, _BC), lambda j: (0, 2 * j)),
                  pl.BlockSpec((64, _BC), lambda j: (0, 2 * j + 1))],
        out_specs=pl.BlockSpec((_BC, 128), lambda j: (j, 0)),
        out_shape=jax.ShapeDtypeStruct((_GRID * _BC, 128), jnp.float32),
    )(wembT, wembT)

    mesh = plsc.VectorSubcoreMesh(core_axis_name="c", subcore_axis_name="s")
    run = pl.kernel(
        _body,
        mesh=mesh,
        compiler_params=pltpu.CompilerParams(
            needs_layout_passes=False, use_tc_tiling_on_sc=False),
        out_type=jax.ShapeDtypeStruct((_BATCH,), jnp.float32),
        scratch_types=[
            pltpu.VMEM((_STEPS * _ROWS_SUB,), jnp.int32),
            pltpu.VMEM((_BPW * _F,), jnp.float32),
            pltpu.VMEM((2, _ROWS_GRP, _EMBED_DIM * 2), jnp.float32),
            pltpu.VMEM((_BPW,), jnp.float32),
            pltpu.VMEM((16,), jnp.float32),
            pltpu.SemaphoreType.DMA,
            pltpu.SemaphoreType.DMA,
        ],
    )
    return run(idx, bias16, w2, wfc_flat)


# speedup vs baseline: 1.7051x; 1.7051x over previous
import functools

import jax
import jax.numpy as jnp
import numpy as np
from jax import lax
from jax.experimental import pallas as pl
from jax.experimental.pallas import tpu as pltpu
from jax.experimental.pallas import tpu_sc as plsc

_FIELD_DIMS = [100000] * 26
_EMBED_DIM = 32
_TOTAL = int(sum(_FIELD_DIMS))
_OFFSETS = np.array((0, *np.cumsum(_FIELD_DIMS)[:-1]), dtype=np.int32)
_BATCH = 4096
_F = len(_FIELD_DIMS)

_NW = 32
_BPW = _BATCH // _NW
_GROUPS = 8
_GB = 16
_SUB = 4
_ROWS_SUB = _GB * _F // _SUB
_ROWS_GRP = _GB * _F
_STEPS = _GROUPS * _SUB


def _body(idx_hbm, bias_hbm, wemb_hbm, wfc_hbm, out_hbm,
          idxf, fcv, ebuf, outv, biasv, sem_e, sem_fc):
    nc = 2
    wid = lax.axis_index("s") * nc + lax.axis_index("c")

    pltpu.sync_copy(idx_hbm.at[wid], idxf)
    pltpu.sync_copy(bias_hbm, biasv)

    for t in range(_STEPS):
        pltpu.async_copy(
            wfc_hbm.at[idxf.at[pl.ds(t * _ROWS_SUB, _ROWS_SUB)]],
            fcv.at[pl.ds(t * _ROWS_SUB, _ROWS_SUB)], sem_fc)

    def _enqueue_group(g, buf):
        def enq(vv, carry):
            off = _ROWS_GRP * g + 16 * vv
            vec = idxf[pl.ds(pl.multiple_of(off, 16), 16)]
            for k in range(16):
                r = vec[k]
                pltpu.async_copy(wemb_hbm.at[r],
                                 ebuf.at[buf, 16 * vv + k], sem_e)
            return carry
        lax.fori_loop(0, _ROWS_GRP // 16, enq, 0)

    def _drain_group(buf):
        pltpu.make_async_copy(wemb_hbm.at[pl.ds(0, _ROWS_GRP)],
                              ebuf.at[buf], sem_e).wait()

    _enqueue_group(0, 0)
    _enqueue_group(1, 1)

    lane = lax.broadcasted_iota(jnp.int32, (16,), 0)

    def _compute_group(g, buf):
        def batch_body(bb, ycarry):
            base = _F * bb
            z = jnp.zeros((16,), jnp.float32)
            s1 = [z, z, z, z]
            s2 = [z, z, z, z]
            s3 = [z, z]
            for f in range(_F):
                for c in range(4):
                    v = ebuf[buf, base + f, pl.ds(16 * c, 16)]
                    s1[c] = s1[c] + v
                    t = v * v
                    s2[c] = s2[c] + t
                    if c >= 2:
                        s3[c - 2] = s3[c - 2] + t * v
            g01 = (s1[0] * s1[0] - s2[0]) + (s1[1] * s1[1] - s2[1])
            h = None
            for c in (2, 3):
                p1, p2, p3 = s1[c], s2[c], s3[c - 2]
                hc = p1 * (p1 * p1 - 3.0 * p2) + 2.0 * p3
                h = hc if h is None else h + hc
            tvec = 0.5 * g01 + (1.0 / 6.0) * h
            y = jnp.sum(tvec)
            ybc = jnp.broadcast_to(y, (16,))
            return jnp.where(lane == bb, ybc, ycarry)
        yg = lax.fori_loop(0, _GB, batch_body, jnp.zeros((16,), jnp.float32))
        outv[pl.ds(pl.multiple_of(_GB * g, 16), _GB)] = yg

    def outer(i, carry):
        g0 = 2 * i
        for buf in range(2):
            g = g0 + buf
            _drain_group(buf)
            _compute_group(g, buf)

            @pl.when(g + 2 < _GROUPS)
            def _():
                _enqueue_group(g + 2, buf)
        return carry

    lax.fori_loop(0, _GROUPS // 2, outer, 0)

    for t in range(_STEPS):
        pltpu.make_async_copy(
            wfc_hbm.at[idxf.at[pl.ds(0, _ROWS_SUB)]],
            fcv.at[pl.ds(t * _ROWS_SUB, _ROWS_SUB)], sem_fc).wait()

    i26 = lax.broadcasted_iota(jnp.int32, (16,), 0) * _F
    for g in range(_GROUPS):
        acc = biasv[...]
        for f in range(_F):
            acc = acc + plsc.load_gather(fcv, [i26 + (_ROWS_GRP * g + f)])
        v = outv[pl.ds(_GB * g, _GB)] + acc
        outv[pl.ds(_GB * g, _GB)] = jnp.maximum(v, 0.0)

    pltpu.sync_copy(outv, out_hbm.at[pl.ds(wid * _BPW, _BPW)])


@functools.partial(jax.jit, static_argnums=())
def kernel(x, W_emb, W_fc, bias):
    offsets = jnp.asarray(_OFFSETS, dtype=jnp.int32)
    idx = (x.astype(jnp.int32) + offsets[None, :]).reshape(
        _NW, _STEPS * _ROWS_SUB)
    bias16 = jnp.broadcast_to(bias.astype(jnp.float32), (16,))
    wfc_flat = W_fc.reshape(_TOTAL)

    mesh = plsc.VectorSubcoreMesh(core_axis_name="c", subcore_axis_name="s")
    run = pl.kernel(
        _body,
        mesh=mesh,
        compiler_params=pltpu.CompilerParams(
            needs_layout_passes=False, use_tc_tiling_on_sc=True),
        out_type=jax.ShapeDtypeStruct((_BATCH,), jnp.float32),
        scratch_types=[
            pltpu.VMEM((_STEPS * _ROWS_SUB,), jnp.int32),
            pltpu.VMEM((_BPW * _F,), jnp.float32),
            pltpu.VMEM((2, _ROWS_GRP, _EMBED_DIM * 2), jnp.float32),
            pltpu.VMEM((_BPW,), jnp.float32),
            pltpu.VMEM((16,), jnp.float32),
            pltpu.SemaphoreType.DMA,
            pltpu.SemaphoreType.DMA,
        ],
    )
    return run(idx, bias16, W_emb, wfc_flat)
